# initial kernel scaffold (unmeasured)
import jax
import jax.numpy as jnp
from jax import lax
from jax.experimental import pallas as pl
from jax.experimental.pallas import tpu as pltpu

N_DEV = 8
_GELU_C = 0.7978845608028654


def kernel(x, w_mat):
    m_tot, k_shard = x.shape
    k_tot, n = w_mat.shape
    m_per = m_tot // N_DEV

    def body(x_ref, w_hbm, out_ref, recv_buf, w_buf, send_sems, recv_sems, w_sems):
        my = lax.axis_index("i")

        def w_copy(t, slot):
            kblk = (my + t) % N_DEV
            return pltpu.make_async_copy(
                w_hbm.at[pl.ds(kblk * m_per, m_per), :],
                w_buf.at[slot],
                w_sems.at[slot],
            )

        w_copy(0, 0).start()

        barrier = pltpu.get_barrier_semaphore()
        for j in range(N_DEV):
            @pl.when(my != j)
            def _():
                pl.semaphore_signal(
                    barrier, inc=1,
                    device_id=(j,), device_id_type=pl.DeviceIdType.MESH,
                )
        pl.semaphore_wait(barrier, N_DEV - 1)

        recv_buf[my, :, :] = x_ref[pl.ds(my * m_per, m_per), :]

        sends = []
        for dj in range(1, N_DEV):
            j = (my + dj) % N_DEV
            rdma = pltpu.make_async_remote_copy(
                src_ref=x_ref.at[pl.ds(j * m_per, m_per), :],
                dst_ref=recv_buf.at[my],
                send_sem=send_sems.at[dj - 1],
                recv_sem=recv_sems.at[my],
                device_id=(j,),
                device_id_type=pl.DeviceIdType.MESH,
            )
            rdma.start()
            sends.append(rdma)

        for t in range(N_DEV):
            j = (my + t) % N_DEV
            slot = t % 2
            if t + 1 < N_DEV:
                w_copy(t + 1, (t + 1) % 2).start()
            w_copy(t, slot).wait()
            if t > 0:
                recv = pltpu.make_async_remote_copy(
                    src_ref=recv_buf.at[j],
                    dst_ref=recv_buf.at[j],
                    send_sem=send_sems.at[0],
                    recv_sem=recv_sems.at[j],
                    device_id=(j,),
                    device_id_type=pl.DeviceIdType.MESH,
                )
                recv.wait_recv()
            prod = lax.dot_general(
                recv_buf[j], w_buf[slot],
                dimension_numbers=(((1,), (0,)), ((), ())),
                preferred_element_type=jnp.float32,
            )
            if t == 0:
                out_ref[...] = prod
            else:
                out_ref[...] += prod

        for rdma in sends:
            rdma.wait_send()

        y = out_ref[...]
        out_ref[...] = 0.5 * y * (1.0 + jnp.tanh(_GELU_C * (y + 0.044715 * y * y * y)))

    return pl.pallas_call(
        body,
        out_shape=jax.ShapeDtypeStruct((m_per, n), jnp.float32),
        in_specs=[
            pl.BlockSpec(memory_space=pltpu.VMEM),
            pl.BlockSpec(memory_space=pltpu.ANY),
        ],
        out_specs=pl.BlockSpec(memory_space=pltpu.VMEM),
        scratch_shapes=[
            pltpu.VMEM((N_DEV, m_per, k_shard), jnp.float32),
            pltpu.VMEM((2, m_per, n), jnp.float32),
            pltpu.SemaphoreType.DMA((N_DEV - 1,)),
            pltpu.SemaphoreType.DMA((N_DEV,)),
            pltpu.SemaphoreType.DMA((2,)),
        ],
        compiler_params=pltpu.CompilerParams(collective_id=0),
    )(x, w_mat)


# baseline (device time: 138881 ns/iter reference)
import jax
import jax.numpy as jnp
from jax import lax
from jax.experimental import pallas as pl
from jax.experimental.pallas import tpu as pltpu

N_DEV = 8
_GELU_C = 0.7978845608028654


def kernel(x, w_mat):
    m_tot, k_shard = x.shape
    k_tot, n = w_mat.shape
    m_per = m_tot // N_DEV

    n_half = n // 2

    def body(x_ref, w_hbm, out_ref, recv_buf, w_buf, send_sems, recv_sems, w_sems):
        my = lax.axis_index("i")

        def w_copy(c, slot):
            kblk = (my + c // 2) % N_DEV
            h = c % 2
            return pltpu.make_async_copy(
                w_hbm.at[pl.ds(kblk * m_per, m_per), pl.ds(h * n_half, n_half)],
                w_buf.at[slot],
                w_sems.at[slot],
            )

        w_copy(0, 0).start()

        barrier = pltpu.get_barrier_semaphore()
        for j in range(N_DEV):
            @pl.when(my != j)
            def _():
                pl.semaphore_signal(
                    barrier, inc=1,
                    device_id=(j,), device_id_type=pl.DeviceIdType.MESH,
                )
        pl.semaphore_wait(barrier, N_DEV - 1)

        recv_buf[my, :, :] = x_ref[pl.ds(my * m_per, m_per), :]

        sends = []
        for dj in range(1, N_DEV):
            j = (my + dj) % N_DEV
            rdma = pltpu.make_async_remote_copy(
                src_ref=x_ref.at[pl.ds(j * m_per, m_per), :],
                dst_ref=recv_buf.at[my],
                send_sem=send_sems.at[dj - 1],
                recv_sem=recv_sems.at[my],
                device_id=(j,),
                device_id_type=pl.DeviceIdType.MESH,
            )
            rdma.start()
            sends.append(rdma)

        n_chunks = 2 * N_DEV
        for c in range(n_chunks):
            t, h = c // 2, c % 2
            j = (my + t) % N_DEV
            slot = c % 2
            if c + 1 < n_chunks:
                w_copy(c + 1, (c + 1) % 2).start()
            w_copy(c, slot).wait()
            if t > 0 and h == 0:
                recv = pltpu.make_async_remote_copy(
                    src_ref=recv_buf.at[j],
                    dst_ref=recv_buf.at[j],
                    send_sem=send_sems.at[0],
                    recv_sem=recv_sems.at[j],
                    device_id=(j,),
                    device_id_type=pl.DeviceIdType.MESH,
                )
                recv.wait_recv()
            prod = lax.dot_general(
                recv_buf[j], w_buf[slot],
                dimension_numbers=(((1,), (0,)), ((), ())),
                preferred_element_type=jnp.float32,
            )
            nd = pl.ds(h * n_half, n_half)
            if t == 0:
                out_ref[:, nd] = prod
            else:
                out_ref[:, nd] += prod

        for rdma in sends:
            rdma.wait_send()

        y = out_ref[...]
        out_ref[...] = 0.5 * y * (1.0 + jnp.tanh(_GELU_C * (y + 0.044715 * y * y * y)))

    return pl.pallas_call(
        body,
        out_shape=jax.ShapeDtypeStruct((m_per, n), jnp.float32),
        in_specs=[
            pl.BlockSpec(memory_space=pltpu.VMEM),
            pl.BlockSpec(memory_space=pl.ANY),
        ],
        out_specs=pl.BlockSpec(memory_space=pltpu.VMEM),
        scratch_shapes=[
            pltpu.VMEM((N_DEV, m_per, k_shard), jnp.float32),
            pltpu.VMEM((2, m_per, n_half), jnp.float32),
            pltpu.SemaphoreType.DMA((N_DEV - 1,)),
            pltpu.SemaphoreType.DMA((N_DEV,)),
            pltpu.SemaphoreType.DMA((2,)),
        ],
        compiler_params=pltpu.CompilerParams(
            collective_id=0,
            vmem_limit_bytes=100 * 1024 * 1024,
        ),
    )(x, w_mat)


# device time: 80586 ns/iter; 1.7234x vs baseline; 1.7234x over previous
import jax
import jax.numpy as jnp
from jax import lax
from jax.experimental import pallas as pl
from jax.experimental.pallas import tpu as pltpu

N_DEV = 8
_GELU_C = 0.7978845608028654

_OFFSETS = [(1, 0, 0), (0, 1, 0), (0, 0, 1), (1, 1, 0), (1, 0, 1), (0, 1, 1), (1, 1, 1)]


def kernel(x, w_mat):
    m_tot, k_shard = x.shape
    k_tot, n = w_mat.shape
    m_per = m_tot // N_DEV
    n_half = n // 2

    def body(x_ref, w_hbm, out_ref, xb16, recv_buf, w_buf,
             send_sems, recv_sems, w_sems):
        my = lax.axis_index("i")

        r = my % 4
        zb = my // 4
        xb = jnp.logical_or(r == 1, r == 2).astype(my.dtype)
        yb = (r >= 2).astype(my.dtype)

        def pos(px, py, pz):
            return 4 * pz + 2 * py + (px ^ py)

        order = [my] + [
            pos(xb ^ dx, yb ^ dy, zb ^ dz) for (dx, dy, dz) in _OFFSETS
        ]

        def w_copy(c, slot):
            kblk = order[c // 2]
            h = c % 2
            return pltpu.make_async_copy(
                w_hbm.at[pl.ds(kblk * m_per, m_per), pl.ds(h * n_half, n_half)],
                w_buf.at[slot],
                w_sems.at[slot],
            )

        w_copy(0, 0).start()

        xb16[...] = x_ref[...].astype(jnp.bfloat16)

        barrier = pltpu.get_barrier_semaphore()
        for j in range(N_DEV):
            @pl.when(my != j)
            def _():
                pl.semaphore_signal(
                    barrier, inc=1,
                    device_id=(j,), device_id_type=pl.DeviceIdType.MESH,
                )
        pl.semaphore_wait(barrier, N_DEV - 1)

        sends = []
        for idx, j in enumerate(order[1:]):
            rdma = pltpu.make_async_remote_copy(
                src_ref=xb16.at[pl.ds(j * m_per, m_per), :],
                dst_ref=recv_buf.at[my],
                send_sem=send_sems.at[idx],
                recv_sem=recv_sems.at[my],
                device_id=(j,),
                device_id_type=pl.DeviceIdType.MESH,
            )
            rdma.start()
            sends.append(rdma)

        n_chunks = 2 * N_DEV
        for c in range(n_chunks):
            t, h = c // 2, c % 2
            j = order[t]
            slot = c % 2
            if c + 1 < n_chunks:
                w_copy(c + 1, (c + 1) % 2).start()
            w_copy(c, slot).wait()
            if t == 0:
                a = xb16[pl.ds(my * m_per, m_per), :]
            else:
                if h == 0:
                    recv = pltpu.make_async_remote_copy(
                        src_ref=recv_buf.at[j],
                        dst_ref=recv_buf.at[j],
                        send_sem=send_sems.at[0],
                        recv_sem=recv_sems.at[j],
                        device_id=(j,),
                        device_id_type=pl.DeviceIdType.MESH,
                    )
                    recv.wait_recv()
                a = recv_buf[j]
            prod = lax.dot_general(
                a.astype(jnp.float32), w_buf[slot],
                dimension_numbers=(((1,), (0,)), ((), ())),
                preferred_element_type=jnp.float32,
            )
            nd = pl.ds(h * n_half, n_half)
            if t == 0:
                out_ref[:, nd] = prod
            else:
                out_ref[:, nd] += prod

        for rdma in sends:
            rdma.wait_send()

        y = out_ref[...]
        out_ref[...] = 0.5 * y * (1.0 + jnp.tanh(_GELU_C * (y + 0.044715 * y * y * y)))

    return pl.pallas_call(
        body,
        out_shape=jax.ShapeDtypeStruct((m_per, n), jnp.float32),
        in_specs=[
            pl.BlockSpec(memory_space=pltpu.VMEM),
            pl.BlockSpec(memory_space=pl.ANY),
        ],
        out_specs=pl.BlockSpec(memory_space=pltpu.VMEM),
        scratch_shapes=[
            pltpu.VMEM((m_tot, k_shard), jnp.bfloat16),
            pltpu.VMEM((N_DEV, m_per, k_shard), jnp.bfloat16),
            pltpu.VMEM((2, m_per, n_half), jnp.float32),
            pltpu.SemaphoreType.DMA((N_DEV - 1,)),
            pltpu.SemaphoreType.DMA((N_DEV,)),
            pltpu.SemaphoreType.DMA((2,)),
        ],
        compiler_params=pltpu.CompilerParams(
            collective_id=0,
            vmem_limit_bytes=100 * 1024 * 1024,
        ),
    )(x, w_mat)


# device time: 80535 ns/iter; 1.7245x vs baseline; 1.0006x over previous
import jax
import jax.numpy as jnp
from jax import lax
from jax.experimental import pallas as pl
from jax.experimental.pallas import tpu as pltpu

N_DEV = 8
_GELU_C = 0.7978845608028654

_OFFSETS = [(1, 0, 0), (0, 1, 0), (0, 0, 1), (1, 1, 0), (1, 0, 1), (0, 1, 1), (1, 1, 1)]


def kernel(x, w_mat):
    m_tot, k_shard = x.shape
    k_tot, n = w_mat.shape
    m_per = m_tot // N_DEV
    n_half = n // 2

    def body(x_ref, w_hbm, out_ref, xb16, recv_buf, w_buf,
             send_sems, recv_sems, w_sems):
        my = lax.axis_index("i")

        r = my % 4
        zb = my // 4
        xb = jnp.logical_or(r == 1, r == 2).astype(my.dtype)
        yb = (r >= 2).astype(my.dtype)

        def pos(px, py, pz):
            return 4 * pz + 2 * py + (px ^ py)

        order = [my] + [
            pos(xb ^ dx, yb ^ dy, zb ^ dz) for (dx, dy, dz) in _OFFSETS
        ]

        def w_copy(c, slot):
            kblk = order[c // 2]
            h = c % 2
            return pltpu.make_async_copy(
                w_hbm.at[pl.ds(kblk * m_per, m_per), pl.ds(h * n_half, n_half)],
                w_buf.at[slot],
                w_sems.at[slot],
            )

        w_copy(0, 0).start()

        xb16[...] = x_ref[...].astype(jnp.bfloat16)

        barrier = pltpu.get_barrier_semaphore()
        for j in range(N_DEV):
            @pl.when(my != j)
            def _():
                pl.semaphore_signal(
                    barrier, inc=1,
                    device_id=(j,), device_id_type=pl.DeviceIdType.MESH,
                )
        pl.semaphore_wait(barrier, N_DEV - 1)

        sends = []

        def send_to(idx, j):
            rdma = pltpu.make_async_remote_copy(
                src_ref=xb16.at[pl.ds(j * m_per, m_per), :],
                dst_ref=recv_buf.at[my],
                send_sem=send_sems.at[idx],
                recv_sem=recv_sems.at[my],
                device_id=(j,),
                device_id_type=pl.DeviceIdType.MESH,
            )
            rdma.start()
            sends.append(rdma)

        for idx in range(3):
            send_to(idx, order[1 + idx])

        n_chunks = 2 * N_DEV
        for c in range(n_chunks):
            if c == 2:
                for idx in range(3, 6):
                    send_to(idx, order[1 + idx])
            if c == 4:
                send_to(6, order[7])
            t, h = c // 2, c % 2
            j = order[t]
            slot = c % 2
            if c + 1 < n_chunks:
                w_copy(c + 1, (c + 1) % 2).start()
            w_copy(c, slot).wait()
            if t == 0:
                a = xb16[pl.ds(my * m_per, m_per), :]
            else:
                if h == 0:
                    recv = pltpu.make_async_remote_copy(
                        src_ref=recv_buf.at[j],
                        dst_ref=recv_buf.at[j],
                        send_sem=send_sems.at[0],
                        recv_sem=recv_sems.at[j],
                        device_id=(j,),
                        device_id_type=pl.DeviceIdType.MESH,
                    )
                    recv.wait_recv()
                a = recv_buf[j]
            prod = lax.dot_general(
                a.astype(jnp.float32), w_buf[slot],
                dimension_numbers=(((1,), (0,)), ((), ())),
                preferred_element_type=jnp.float32,
            )
            nd = pl.ds(h * n_half, n_half)
            if t == 0:
                out_ref[:, nd] = prod
            else:
                out_ref[:, nd] += prod

        for rdma in sends:
            rdma.wait_send()

        y = out_ref[...]
        out_ref[...] = 0.5 * y * (1.0 + jnp.tanh(_GELU_C * (y + 0.044715 * y * y * y)))

    return pl.pallas_call(
        body,
        out_shape=jax.ShapeDtypeStruct((m_per, n), jnp.float32),
        in_specs=[
            pl.BlockSpec(memory_space=pltpu.VMEM),
            pl.BlockSpec(memory_space=pl.ANY),
        ],
        out_specs=pl.BlockSpec(memory_space=pltpu.VMEM),
        scratch_shapes=[
            pltpu.VMEM((m_tot, k_shard), jnp.bfloat16),
            pltpu.VMEM((N_DEV, m_per, k_shard), jnp.bfloat16),
            pltpu.VMEM((2, m_per, n_half), jnp.float32),
            pltpu.SemaphoreType.DMA((N_DEV - 1,)),
            pltpu.SemaphoreType.DMA((N_DEV,)),
            pltpu.SemaphoreType.DMA((2,)),
        ],
        compiler_params=pltpu.CompilerParams(
            collective_id=0,
            vmem_limit_bytes=100 * 1024 * 1024,
        ),
    )(x, w_mat)


# device time: 79915 ns/iter; 1.7379x vs baseline; 1.0078x over previous
import jax
import jax.numpy as jnp
from jax import lax
from jax.experimental import pallas as pl
from jax.experimental.pallas import tpu as pltpu

N_DEV = 8
_GELU_C = 0.7978845608028654

_OFFSETS = [(1, 0, 0), (0, 1, 0), (0, 0, 1), (1, 1, 0), (1, 0, 1), (0, 1, 1), (1, 1, 1)]


def kernel(x, w_mat):
    m_tot, k_shard = x.shape
    k_tot, n = w_mat.shape
    m_per = m_tot // N_DEV
    n_half = n // 2

    def body(x_ref, w_hbm, out_ref, xb16, recv_buf, af32, w_buf,
             send_sems, recv_sems, w_sems):
        my = lax.axis_index("i")

        r = my % 4
        zb = my // 4
        xb = jnp.logical_or(r == 1, r == 2).astype(my.dtype)
        yb = (r >= 2).astype(my.dtype)

        def pos(px, py, pz):
            return 4 * pz + 2 * py + (px ^ py)

        order = [my] + [
            pos(xb ^ dx, yb ^ dy, zb ^ dz) for (dx, dy, dz) in _OFFSETS
        ]

        def w_copy(c, slot):
            kblk = order[c // 2]
            h = c % 2
            return pltpu.make_async_copy(
                w_hbm.at[pl.ds(kblk * m_per, m_per), pl.ds(h * n_half, n_half)],
                w_buf.at[slot],
                w_sems.at[slot],
            )

        w_copy(0, 0).start()

        xb16[...] = x_ref[...].astype(jnp.bfloat16)

        barrier = pltpu.get_barrier_semaphore()
        for j in range(N_DEV):
            @pl.when(my != j)
            def _():
                pl.semaphore_signal(
                    barrier, inc=1,
                    device_id=(j,), device_id_type=pl.DeviceIdType.MESH,
                )
        pl.semaphore_wait(barrier, N_DEV - 1)

        sends = []

        def send_to(idx, j):
            rdma = pltpu.make_async_remote_copy(
                src_ref=xb16.at[pl.ds(j * m_per, m_per), :],
                dst_ref=recv_buf.at[my],
                send_sem=send_sems.at[idx],
                recv_sem=recv_sems.at[my],
                device_id=(j,),
                device_id_type=pl.DeviceIdType.MESH,
            )
            rdma.start()
            sends.append(rdma)

        for idx in range(3):
            send_to(idx, order[1 + idx])

        n_chunks = 2 * N_DEV
        for c in range(n_chunks):
            if c == 2:
                for idx in range(3, 6):
                    send_to(idx, order[1 + idx])
            if c == 4:
                send_to(6, order[7])
            t, h = c // 2, c % 2
            j = order[t]
            slot = c % 2
            if c + 1 < n_chunks:
                w_copy(c + 1, (c + 1) % 2).start()
            w_copy(c, slot).wait()
            if h == 0:
                if t == 0:
                    af32[...] = xb16[pl.ds(my * m_per, m_per), :].astype(jnp.float32)
                else:
                    recv = pltpu.make_async_remote_copy(
                        src_ref=recv_buf.at[j],
                        dst_ref=recv_buf.at[j],
                        send_sem=send_sems.at[0],
                        recv_sem=recv_sems.at[j],
                        device_id=(j,),
                        device_id_type=pl.DeviceIdType.MESH,
                    )
                    recv.wait_recv()
                    af32[...] = recv_buf[j].astype(jnp.float32)
            prod = lax.dot_general(
                af32[...], w_buf[slot],
                dimension_numbers=(((1,), (0,)), ((), ())),
                preferred_element_type=jnp.float32,
            )
            nd = pl.ds(h * n_half, n_half)
            if t == 0:
                out_ref[:, nd] = prod
            else:
                out_ref[:, nd] += prod
                if t == N_DEV - 1:
                    y = out_ref[:, nd]
                    out_ref[:, nd] = 0.5 * y * (
                        1.0 + jnp.tanh(_GELU_C * (y + 0.044715 * y * y * y))
                    )

        for rdma in sends:
            rdma.wait_send()

    return pl.pallas_call(
        body,
        out_shape=jax.ShapeDtypeStruct((m_per, n), jnp.float32),
        in_specs=[
            pl.BlockSpec(memory_space=pltpu.VMEM),
            pl.BlockSpec(memory_space=pl.ANY),
        ],
        out_specs=pl.BlockSpec(memory_space=pltpu.VMEM),
        scratch_shapes=[
            pltpu.VMEM((m_tot, k_shard), jnp.bfloat16),
            pltpu.VMEM((N_DEV, m_per, k_shard), jnp.bfloat16),
            pltpu.VMEM((m_per, k_shard), jnp.float32),
            pltpu.VMEM((2, m_per, n_half), jnp.float32),
            pltpu.SemaphoreType.DMA((N_DEV - 1,)),
            pltpu.SemaphoreType.DMA((N_DEV,)),
            pltpu.SemaphoreType.DMA((2,)),
        ],
        compiler_params=pltpu.CompilerParams(
            collective_id=0,
            vmem_limit_bytes=100 * 1024 * 1024,
        ),
    )(x, w_mat)
